# Initial kernel scaffold; baseline (speedup 1.0000x reference)
import jax
import jax.numpy as jnp
from jax.experimental import pallas as pl


def kernel(z, W1, b1, ln_g, ln_b, W2, b2, competence, activation_ema):
    raise NotImplementedError("write your pallas kernel here")


# speedup vs baseline: 8.3144x; 8.3144x over previous
import functools

import jax
import jax.numpy as jnp
import numpy as np
from jax import lax
from jax.experimental import pallas as pl
from jax.experimental.pallas import tpu as pltpu
from jax.experimental.pallas import tpu_sc as plsc

_B = 32
_H = 1024
_HH = 512
_N = 100000
_K = 64
_BN = 2048
_NBLK = (_N + _BN - 1) // _BN

_NP = _NBLK * _BN
_CH = 256
_G = _NP // _CH
_GP = 400
_J = _GP // 16


def _scores_body(z_ref, w1_ref, b1_ref, g_ref, bb_ref, w2_ref, b2_ref,
                 comp_ref, ema_ref, out_ref, h_ref):
    @pl.when(pl.program_id(0) == 0)
    def _():
        h = jnp.dot(z_ref[...], w1_ref[...],
                    preferred_element_type=jnp.float32) + b1_ref[...]
        mu = jnp.mean(h, axis=-1, keepdims=True)
        var = jnp.mean((h - mu) ** 2, axis=-1, keepdims=True)
        h = (h - mu) / jnp.sqrt(var + 1e-5) * g_ref[...] + bb_ref[...]
        h_ref[...] = h * 0.5 * (1.0 + lax.erf(h / np.sqrt(2.0).astype(np.float32)))

    s = jnp.dot(h_ref[...], w2_ref[...],
                preferred_element_type=jnp.float32) + b2_ref[...]
    s = s + comp_ref[...] * 0.3 + (1.0 / (ema_ref[...] + 1e-6)) * 0.1
    col = pl.program_id(0) * _BN + lax.broadcasted_iota(jnp.int32, (_B, _BN), 1)
    out_ref[...] = jnp.where(col < _N, s, -jnp.inf)


def _tc_scores(z, W1, b1, ln_g, ln_b, W2, b2, comp, ema):
    return pl.pallas_call(
        _scores_body,
        grid=(_NBLK,),
        in_specs=[
            pl.BlockSpec((_B, _H), lambda j: (0, 0)),
            pl.BlockSpec((_H, _HH), lambda j: (0, 0)),
            pl.BlockSpec((1, _HH), lambda j: (0, 0)),
            pl.BlockSpec((1, _HH), lambda j: (0, 0)),
            pl.BlockSpec((1, _HH), lambda j: (0, 0)),
            pl.BlockSpec((_HH, _BN), lambda j: (0, j)),
            pl.BlockSpec((1, _BN), lambda j: (0, j)),
            pl.BlockSpec((1, _BN), lambda j: (0, j)),
            pl.BlockSpec((1, _BN), lambda j: (0, j)),
        ],
        out_specs=pl.BlockSpec((_B, _BN), lambda j: (0, j)),
        out_shape=jax.ShapeDtypeStruct((_B, _NP), jnp.float32),
        scratch_shapes=[pltpu.VMEM((_B, _HH), jnp.float32)],
    )(z, W1, b1, ln_g, ln_b, W2, b2, comp, ema)


_GDN = lax.GatherDimensionNumbers(offset_dims=(), collapsed_slice_dims=(0,),
                                  start_index_map=(0,))


def _shuf(x, idx):
    return lax.gather(x, idx[:, None], dimension_numbers=_GDN,
                      slice_sizes=(1,),
                      mode=lax.GatherScatterMode.PROMISE_IN_BOUNDS)


def _pair_max(m, mi, x, xi):
    upd = x > m
    return jnp.where(upd, x, m), jnp.where(upd, xi, mi)


def _cross_lane_argmax(lane, v, i):
    for s in (8, 4, 2, 1):
        p = lane ^ s
        zv = _shuf(v, p)
        zi = _shuf(i, p)
        take = jnp.logical_or(zv > v, jnp.logical_and(zv == v, zi < i))
        v = jnp.where(take, zv, v)
        i = jnp.where(take, zi, i)
    return v[0], i[0]


def _sc_body(scores_hbm, mask_hbm, sel_hbm, row_v, m1_v, m2v_v, m2i_v, idx_v):
    row = lax.axis_index("s") * 2 + lax.axis_index("c")
    neg = jnp.float32(-jnp.inf)
    neg_v = jnp.full((16,), neg, jnp.float32)
    lane = lax.broadcasted_iota(jnp.int32, (16,), 0)

    pltpu.sync_copy(scores_hbm.at[pl.ds(row * _NP, _NP)], row_v)

    def m1_body(g, c):
        base = g * _CH
        m = row_v[pl.ds(base, 16)]
        for k in range(1, 16):
            m = jnp.maximum(m, row_v[pl.ds(base + k * 16, 16)])
        m1_v[pl.ds(g * 16, 16)] = m
        return c

    lax.fori_loop(0, _G, m1_body, 0)
    for g in range(_G, _GP):
        m1_v[pl.ds(g * 16, 16)] = neg_v

    def m2_body(j, c):
        m = m1_v[pl.ds(j * 256, 16)]
        mi = jnp.full((16,), 0, jnp.int32) + j * 16
        for k in range(1, 16):
            m, mi = _pair_max(m, mi, m1_v[pl.ds(j * 256 + k * 16, 16)],
                              jnp.full((16,), 0, jnp.int32) + (j * 16 + k))
        m2v_v[pl.ds(j * 16, 16)] = m
        m2i_v[pl.ds(j * 16, 16)] = mi
        return c

    lax.fori_loop(0, _J, m2_body, 0)

    def extract(t, c):
        m = m2v_v[pl.ds(0, 16)]
        mi = m2i_v[pl.ds(0, 16)]
        for j in range(1, _J):
            m, mi = _pair_max(m, mi, m2v_v[pl.ds(j * 16, 16)],
                              m2i_v[pl.ds(j * 16, 16)])
        v, gbest = _cross_lane_argmax(lane, m, mi)
        vs = jnp.full((16,), v, jnp.float32)

        kb0 = gbest * _CH
        em = row_v[pl.ds(kb0, 16)]
        ei = kb0 + lane
        for k in range(1, 16):
            em, ei = _pair_max(em, ei, row_v[pl.ds(kb0 + k * 16, 16)],
                               kb0 + k * 16 + lane)
        _, flat = _cross_lane_argmax(lane, em, ei)

        slot = (t >> 4) << 4
        iv = idx_v[pl.ds(slot, 16)]
        idx_v[pl.ds(slot, 16)] = jnp.where(lane == (t & 15), flat, iv)

        kb = (flat >> 4) << 4
        vreg = row_v[pl.ds(kb, 16)]
        row_v[pl.ds(kb, 16)] = jnp.where(lane == (flat & 15), neg, vreg)

        m = row_v[pl.ds(kb0, 16)]
        for k in range(1, 16):
            m = jnp.maximum(m, row_v[pl.ds(kb0 + k * 16, 16)])
        m1_v[pl.ds(gbest * 16, 16)] = m

        jb = gbest >> 4
        m = m1_v[pl.ds(jb * 256, 16)]
        mi = jnp.full((16,), 0, jnp.int32) + jb * 16
        for k in range(1, 16):
            m, mi = _pair_max(m, mi, m1_v[pl.ds(jb * 256 + k * 16, 16)],
                              jb * 16 + k + jnp.full((16,), 0, jnp.int32))
        m2v_v[pl.ds(jb * 16, 16)] = m
        m2i_v[pl.ds(jb * 16, 16)] = mi
        return c

    lax.fori_loop(0, _K, extract, 0)

    zero_v = jnp.zeros((16,), jnp.float32)

    def zero_body(i, c):
        row_v[pl.ds(i * 16, 16)] = zero_v
        return c

    lax.fori_loop(0, _NP // 16, zero_body, 0)

    def ones_body(t, c):
        iv = idx_v[pl.ds((t >> 4) << 4, 16)]
        m = jnp.where(lane == (t & 15), iv, jnp.int32(-2147483648))
        for s in (8, 4, 2, 1):
            m = jnp.maximum(m, _shuf(m, lane ^ s))
        flat = m[0]
        kb = (flat >> 4) << 4
        vreg = row_v[pl.ds(kb, 16)]
        row_v[pl.ds(kb, 16)] = jnp.where(lane == (flat & 15),
                                         jnp.float32(1.0), vreg)
        return c

    lax.fori_loop(0, _K, ones_body, 0)
    pltpu.sync_copy(row_v, mask_hbm.at[pl.ds(row * _NP, _NP)])

    @pl.when(row == 0)
    def _():
        pltpu.sync_copy(idx_v, sel_hbm)


@functools.lru_cache(maxsize=1)
def _sc_topk():
    return pl.kernel(
        _sc_body,
        out_type=(jax.ShapeDtypeStruct((_B * _NP,), jnp.float32),
                  jax.ShapeDtypeStruct((_K,), jnp.int32)),
        mesh=plsc.VectorSubcoreMesh(core_axis_name="c", subcore_axis_name="s",
                                    num_cores=2, num_subcores=16),
        scratch_types=[
            pltpu.VMEM((_NP,), jnp.float32),
            pltpu.VMEM((_GP * 16,), jnp.float32),
            pltpu.VMEM((_J * 16,), jnp.float32),
            pltpu.VMEM((_J * 16,), jnp.int32),
            pltpu.VMEM((_K,), jnp.int32),
        ],
    )


def kernel(z, W1, b1, ln_g, ln_b, W2, b2, competence, activation_ema):
    padded = _tc_scores(z, W1, b1.reshape(1, _HH), ln_g.reshape(1, _HH),
                        ln_b.reshape(1, _HH), W2, b2.reshape(1, _N),
                        competence.reshape(1, _N),
                        activation_ema.reshape(1, _N))
    mask_flat, sel = _sc_topk()(padded.reshape(_B * _NP))
    mask = mask_flat.reshape(_B, _NP)[:, :_N]
    return mask, sel, padded[:, :_N]


# speedup vs baseline: 8.4238x; 1.0132x over previous
import functools

import jax
import jax.numpy as jnp
import numpy as np
from jax import lax
from jax.experimental import pallas as pl
from jax.experimental.pallas import tpu as pltpu
from jax.experimental.pallas import tpu_sc as plsc

_B = 32
_H = 1024
_HH = 512
_N = 100000
_K = 64
_BN = 8192
_NBLK = (_N + _BN - 1) // _BN

_NP = _NBLK * _BN
_CH = 256
_G = _NP // _CH
_GP = _G
_J = _GP // 16


def _scores_body(z_ref, w1_ref, b1_ref, g_ref, bb_ref, w2_ref, b2_ref,
                 comp_ref, ema_ref, out_ref, exact_ref, h_ref):
    @pl.when(pl.program_id(0) == 0)
    def _():
        h = jnp.dot(z_ref[...], w1_ref[...],
                    preferred_element_type=jnp.float32) + b1_ref[...]
        mu = jnp.mean(h, axis=-1, keepdims=True)
        var = jnp.mean((h - mu) ** 2, axis=-1, keepdims=True)
        h = (h - mu) / jnp.sqrt(var + 1e-5) * g_ref[...] + bb_ref[...]
        h_ref[...] = h * 0.5 * (1.0 + lax.erf(h / np.sqrt(2.0).astype(np.float32)))

    s = jnp.dot(h_ref[...], w2_ref[...],
                preferred_element_type=jnp.float32) + b2_ref[...]
    s = s + comp_ref[...] * 0.3 + (1.0 / (ema_ref[...] + 1e-6)) * 0.1
    col = pl.program_id(0) * _BN + lax.broadcasted_iota(jnp.int32, (_B, _BN), 1)
    out_ref[...] = jnp.where(col < _N, s, -jnp.inf)
    exact_ref[...] = s


def _tc_scores(z, W1, b1, ln_g, ln_b, W2, b2, comp, ema):
    return pl.pallas_call(
        _scores_body,
        grid=(_NBLK,),
        in_specs=[
            pl.BlockSpec((_B, _H), lambda j: (0, 0)),
            pl.BlockSpec((_H, _HH), lambda j: (0, 0)),
            pl.BlockSpec((1, _HH), lambda j: (0, 0)),
            pl.BlockSpec((1, _HH), lambda j: (0, 0)),
            pl.BlockSpec((1, _HH), lambda j: (0, 0)),
            pl.BlockSpec((_HH, _BN), lambda j: (0, j)),
            pl.BlockSpec((1, _BN), lambda j: (0, j)),
            pl.BlockSpec((1, _BN), lambda j: (0, j)),
            pl.BlockSpec((1, _BN), lambda j: (0, j)),
        ],
        out_specs=[pl.BlockSpec((_B, _BN), lambda j: (0, j)),
                   pl.BlockSpec((_B, _BN), lambda j: (0, j))],
        out_shape=[jax.ShapeDtypeStruct((_B, _NP), jnp.float32),
                   jax.ShapeDtypeStruct((_B, _N), jnp.float32)],
        scratch_shapes=[pltpu.VMEM((_B, _HH), jnp.float32)],
    )(z, W1, b1, ln_g, ln_b, W2, b2, comp, ema)


_GDN = lax.GatherDimensionNumbers(offset_dims=(), collapsed_slice_dims=(0,),
                                  start_index_map=(0,))


def _shuf(x, idx):
    return lax.gather(x, idx[:, None], dimension_numbers=_GDN,
                      slice_sizes=(1,),
                      mode=lax.GatherScatterMode.PROMISE_IN_BOUNDS)


def _pair_max(m, mi, x, xi):
    upd = x > m
    return jnp.where(upd, x, m), jnp.where(upd, xi, mi)


def _cross_lane_argmax(lane, v, i):
    for s in (8, 4, 2, 1):
        p = lane ^ s
        zv = _shuf(v, p)
        zi = _shuf(i, p)
        take = jnp.logical_or(zv > v, jnp.logical_and(zv == v, zi < i))
        v = jnp.where(take, zv, v)
        i = jnp.where(take, zi, i)
    return v[0], i[0]


def _sc_body(scores_hbm, mask_hbm, sel_hbm, row_v, m1_v, m2v_v, m2i_v, idx_v):
    row = lax.axis_index("s") * 2 + lax.axis_index("c")
    neg = jnp.float32(-jnp.inf)
    neg_v = jnp.full((16,), neg, jnp.float32)
    lane = lax.broadcasted_iota(jnp.int32, (16,), 0)

    pltpu.sync_copy(scores_hbm.at[pl.ds(row * _NP, _NP)], row_v)

    def m1_body(g, c):
        base = g * _CH
        m = row_v[pl.ds(base, 16)]
        for k in range(1, 16):
            m = jnp.maximum(m, row_v[pl.ds(base + k * 16, 16)])
        m1_v[pl.ds(g * 16, 16)] = m
        return c

    lax.fori_loop(0, _G, m1_body, 0)
    for g in range(_G, _GP):
        m1_v[pl.ds(g * 16, 16)] = neg_v

    def m2_body(j, c):
        m = m1_v[pl.ds(j * 256, 16)]
        mi = jnp.full((16,), 0, jnp.int32) + j * 16
        for k in range(1, 16):
            m, mi = _pair_max(m, mi, m1_v[pl.ds(j * 256 + k * 16, 16)],
                              jnp.full((16,), 0, jnp.int32) + (j * 16 + k))
        m2v_v[pl.ds(j * 16, 16)] = m
        m2i_v[pl.ds(j * 16, 16)] = mi
        return c

    lax.fori_loop(0, _J, m2_body, 0)

    def extract(t, c):
        m = m2v_v[pl.ds(0, 16)]
        mi = m2i_v[pl.ds(0, 16)]
        for j in range(1, _J):
            m, mi = _pair_max(m, mi, m2v_v[pl.ds(j * 16, 16)],
                              m2i_v[pl.ds(j * 16, 16)])
        v, gbest = _cross_lane_argmax(lane, m, mi)
        vs = jnp.full((16,), v, jnp.float32)

        kb0 = gbest * _CH
        em = row_v[pl.ds(kb0, 16)]
        ei = kb0 + lane
        for k in range(1, 16):
            em, ei = _pair_max(em, ei, row_v[pl.ds(kb0 + k * 16, 16)],
                               kb0 + k * 16 + lane)
        _, flat = _cross_lane_argmax(lane, em, ei)

        slot = (t >> 4) << 4
        iv = idx_v[pl.ds(slot, 16)]
        idx_v[pl.ds(slot, 16)] = jnp.where(lane == (t & 15), flat, iv)

        kb = (flat >> 4) << 4
        vreg = row_v[pl.ds(kb, 16)]
        row_v[pl.ds(kb, 16)] = jnp.where(lane == (flat & 15), neg, vreg)

        m = row_v[pl.ds(kb0, 16)]
        for k in range(1, 16):
            m = jnp.maximum(m, row_v[pl.ds(kb0 + k * 16, 16)])
        m1_v[pl.ds(gbest * 16, 16)] = m

        jb = gbest >> 4
        m = m1_v[pl.ds(jb * 256, 16)]
        mi = jnp.full((16,), 0, jnp.int32) + jb * 16
        for k in range(1, 16):
            m, mi = _pair_max(m, mi, m1_v[pl.ds(jb * 256 + k * 16, 16)],
                              jb * 16 + k + jnp.full((16,), 0, jnp.int32))
        m2v_v[pl.ds(jb * 16, 16)] = m
        m2i_v[pl.ds(jb * 16, 16)] = mi
        return c

    lax.fori_loop(0, _K, extract, 0)

    zero_v = jnp.zeros((16,), jnp.float32)

    def zero_body(i, c):
        row_v[pl.ds(i * 16, 16)] = zero_v
        return c

    lax.fori_loop(0, _NP // 16, zero_body, 0)

    def ones_body(t, c):
        iv = idx_v[pl.ds((t >> 4) << 4, 16)]
        m = jnp.where(lane == (t & 15), iv, jnp.int32(-2147483648))
        for s in (8, 4, 2, 1):
            m = jnp.maximum(m, _shuf(m, lane ^ s))
        flat = m[0]
        kb = (flat >> 4) << 4
        vreg = row_v[pl.ds(kb, 16)]
        row_v[pl.ds(kb, 16)] = jnp.where(lane == (flat & 15),
                                         jnp.float32(1.0), vreg)
        return c

    lax.fori_loop(0, _K, ones_body, 0)
    pltpu.sync_copy(row_v, mask_hbm.at[pl.ds(row * _NP, _NP)])

    @pl.when(row == 0)
    def _():
        pltpu.sync_copy(idx_v, sel_hbm)


@functools.lru_cache(maxsize=1)
def _sc_topk():
    return pl.kernel(
        _sc_body,
        out_type=(jax.ShapeDtypeStruct((_B * _NP,), jnp.float32),
                  jax.ShapeDtypeStruct((_K,), jnp.int32)),
        mesh=plsc.VectorSubcoreMesh(core_axis_name="c", subcore_axis_name="s",
                                    num_cores=2, num_subcores=16),
        scratch_types=[
            pltpu.VMEM((_NP,), jnp.float32),
            pltpu.VMEM((_GP * 16,), jnp.float32),
            pltpu.VMEM((_J * 16,), jnp.float32),
            pltpu.VMEM((_J * 16,), jnp.int32),
            pltpu.VMEM((_K,), jnp.int32),
        ],
    )


def kernel(z, W1, b1, ln_g, ln_b, W2, b2, competence, activation_ema):
    padded, exact = _tc_scores(z, W1, b1.reshape(1, _HH), ln_g.reshape(1, _HH),
                               ln_b.reshape(1, _HH), W2, b2.reshape(1, _N),
                               competence.reshape(1, _N),
                               activation_ema.reshape(1, _N))
    mask_flat, sel = _sc_topk()(padded.reshape(_B * _NP))
    mask = mask_flat.reshape(_B, _NP)[:, :_N]
    return mask, sel, exact


# speedup vs baseline: 8.9925x; 1.0675x over previous
import functools

import jax
import jax.numpy as jnp
import numpy as np
from jax import lax
from jax.experimental import pallas as pl
from jax.experimental.pallas import tpu as pltpu
from jax.experimental.pallas import tpu_sc as plsc

_B = 32
_H = 1024
_HH = 512
_N = 100000
_K = 64
_BN = 8192
_NBLK = (_N + _BN - 1) // _BN

_NP = _NBLK * _BN
_CH = 256
_G = _NP // _CH
_GP = _G
_J = _GP // 16


def _scores_body(z_ref, w1_ref, b1_ref, g_ref, bb_ref, w2_ref, b2_ref,
                 comp_ref, ema_ref, out_ref, exact_ref, h_ref):
    @pl.when(pl.program_id(0) == 0)
    def _():
        h = jnp.dot(z_ref[...], w1_ref[...],
                    preferred_element_type=jnp.float32) + b1_ref[...]
        mu = jnp.mean(h, axis=-1, keepdims=True)
        var = jnp.mean((h - mu) ** 2, axis=-1, keepdims=True)
        h = (h - mu) / jnp.sqrt(var + 1e-5) * g_ref[...] + bb_ref[...]
        h_ref[...] = h * 0.5 * (1.0 + lax.erf(h / np.sqrt(2.0).astype(np.float32)))

    s = jnp.dot(h_ref[...], w2_ref[...],
                preferred_element_type=jnp.float32) + b2_ref[...]
    s = s + comp_ref[...] * 0.3 + (1.0 / (ema_ref[...] + 1e-6)) * 0.1
    col = pl.program_id(0) * _BN + lax.broadcasted_iota(jnp.int32, (_B, _BN), 1)
    out_ref[...] = jnp.where(col < _N, s, -jnp.inf)
    exact_ref[...] = s


def _tc_scores(z, W1, b1, ln_g, ln_b, W2, b2, comp, ema):
    return pl.pallas_call(
        _scores_body,
        grid=(_NBLK,),
        in_specs=[
            pl.BlockSpec((_B, _H), lambda j: (0, 0)),
            pl.BlockSpec((_H, _HH), lambda j: (0, 0)),
            pl.BlockSpec((1, _HH), lambda j: (0, 0)),
            pl.BlockSpec((1, _HH), lambda j: (0, 0)),
            pl.BlockSpec((1, _HH), lambda j: (0, 0)),
            pl.BlockSpec((_HH, _BN), lambda j: (0, j)),
            pl.BlockSpec((1, _BN), lambda j: (0, j)),
            pl.BlockSpec((1, _BN), lambda j: (0, j)),
            pl.BlockSpec((1, _BN), lambda j: (0, j)),
        ],
        out_specs=[pl.BlockSpec((_B, _BN), lambda j: (0, j)),
                   pl.BlockSpec((_B, _BN), lambda j: (0, j))],
        out_shape=[jax.ShapeDtypeStruct((_B, _NP), jnp.float32),
                   jax.ShapeDtypeStruct((_B, _N), jnp.float32)],
        scratch_shapes=[pltpu.VMEM((_B, _HH), jnp.float32)],
    )(z, W1, b1, ln_g, ln_b, W2, b2, comp, ema)


_GDN = lax.GatherDimensionNumbers(offset_dims=(), collapsed_slice_dims=(0,),
                                  start_index_map=(0,))


def _shuf(x, idx):
    return lax.gather(x, idx[:, None], dimension_numbers=_GDN,
                      slice_sizes=(1,),
                      mode=lax.GatherScatterMode.PROMISE_IN_BOUNDS)


def _pair_max(m, mi, x, xi):
    upd = x > m
    return jnp.where(upd, x, m), jnp.where(upd, xi, mi)


def _cross_lane_argmax(lane, v, i):
    for s in (8, 4, 2, 1):
        p = lane ^ s
        zv = _shuf(v, p)
        zi = _shuf(i, p)
        take = jnp.logical_or(zv > v, jnp.logical_and(zv == v, zi < i))
        v = jnp.where(take, zv, v)
        i = jnp.where(take, zi, i)
    return v[0], i[0]


def _sc_body(scores_hbm, mask_hbm, sel_hbm, row_v, m1_v, m2v_v, m2i_v, idx_v):
    row = lax.axis_index("s") * 2 + lax.axis_index("c")
    neg = jnp.float32(-jnp.inf)
    neg_v = jnp.full((16,), neg, jnp.float32)
    lane = lax.broadcasted_iota(jnp.int32, (16,), 0)

    pltpu.sync_copy(scores_hbm.at[pl.ds(row * _NP, _NP)], row_v)

    def m1_body(g, c):
        base = g * _CH
        m = row_v[pl.ds(base, 16)]
        for k in range(1, 16):
            m = jnp.maximum(m, row_v[pl.ds(base + k * 16, 16)])
        m1_v[pl.ds(g * 16, 16)] = m
        return c

    lax.fori_loop(0, _G, m1_body, 0)
    for g in range(_G, _GP):
        m1_v[pl.ds(g * 16, 16)] = neg_v

    def m2_body(j, c):
        m = m1_v[pl.ds(j * 256, 16)]
        mi = jnp.full((16,), 0, jnp.int32) + j * 16
        for k in range(1, 16):
            m, mi = _pair_max(m, mi, m1_v[pl.ds(j * 256 + k * 16, 16)],
                              jnp.full((16,), 0, jnp.int32) + (j * 16 + k))
        m2v_v[pl.ds(j * 16, 16)] = m
        m2i_v[pl.ds(j * 16, 16)] = mi
        return c

    lax.fori_loop(0, _J, m2_body, 0)

    def extract(t, c):
        m = m2v_v[pl.ds(0, 16)]
        mi = m2i_v[pl.ds(0, 16)]
        for j in range(1, _J):
            m, mi = _pair_max(m, mi, m2v_v[pl.ds(j * 16, 16)],
                              m2i_v[pl.ds(j * 16, 16)])
        v, gbest = _cross_lane_argmax(lane, m, mi)
        vs = jnp.full((16,), v, jnp.float32)

        kb0 = gbest * _CH
        em = row_v[pl.ds(kb0, 16)]
        ei = kb0 + lane
        for k in range(1, 16):
            em, ei = _pair_max(em, ei, row_v[pl.ds(kb0 + k * 16, 16)],
                               kb0 + k * 16 + lane)
        _, flat = _cross_lane_argmax(lane, em, ei)

        slot = (t >> 4) << 4
        iv = idx_v[pl.ds(slot, 16)]
        idx_v[pl.ds(slot, 16)] = jnp.where(lane == (t & 15), flat, iv)

        kb = (flat >> 4) << 4
        vreg = row_v[pl.ds(kb, 16)]
        row_v[pl.ds(kb, 16)] = jnp.where(lane == (flat & 15), neg, vreg)

        m = row_v[pl.ds(kb0, 16)]
        for k in range(1, 16):
            m = jnp.maximum(m, row_v[pl.ds(kb0 + k * 16, 16)])
        m1_v[pl.ds(gbest * 16, 16)] = m

        jb = gbest >> 4
        m = m1_v[pl.ds(jb * 256, 16)]
        mi = jnp.full((16,), 0, jnp.int32) + jb * 16
        for k in range(1, 16):
            m, mi = _pair_max(m, mi, m1_v[pl.ds(jb * 256 + k * 16, 16)],
                              jb * 16 + k + jnp.full((16,), 0, jnp.int32))
        m2v_v[pl.ds(jb * 16, 16)] = m
        m2i_v[pl.ds(jb * 16, 16)] = mi
        return c

    lax.fori_loop(0, _K, extract, 0)

    zero_v = jnp.zeros((16,), jnp.float32)

    def zero_body(i, c):
        base = i * _CH
        for k in range(16):
            row_v[pl.ds(base + k * 16, 16)] = zero_v
        return c

    lax.fori_loop(0, _NP // _CH, zero_body, 0)

    def ones_body(t, c):
        iv = idx_v[pl.ds((t >> 4) << 4, 16)]
        m = jnp.where(lane == (t & 15), iv, jnp.int32(-2147483648))
        for s in (8, 4, 2, 1):
            m = jnp.maximum(m, _shuf(m, lane ^ s))
        flat = m[0]
        kb = (flat >> 4) << 4
        vreg = row_v[pl.ds(kb, 16)]
        row_v[pl.ds(kb, 16)] = jnp.where(lane == (flat & 15),
                                         jnp.float32(1.0), vreg)
        return c

    lax.fori_loop(0, _K, ones_body, 0)
    pltpu.sync_copy(row_v, mask_hbm.at[pl.ds(row * _NP, _NP)])

    @pl.when(row == 0)
    def _():
        pltpu.sync_copy(idx_v, sel_hbm)


@functools.lru_cache(maxsize=1)
def _sc_topk():
    return pl.kernel(
        _sc_body,
        out_type=(jax.ShapeDtypeStruct((_B * _NP,), jnp.float32),
                  jax.ShapeDtypeStruct((_K,), jnp.int32)),
        mesh=plsc.VectorSubcoreMesh(core_axis_name="c", subcore_axis_name="s",
                                    num_cores=2, num_subcores=16),
        scratch_types=[
            pltpu.VMEM((_NP,), jnp.float32),
            pltpu.VMEM((_GP * 16,), jnp.float32),
            pltpu.VMEM((_J * 16,), jnp.float32),
            pltpu.VMEM((_J * 16,), jnp.int32),
            pltpu.VMEM((_K,), jnp.int32),
        ],
    )


def kernel(z, W1, b1, ln_g, ln_b, W2, b2, competence, activation_ema):
    padded, exact = _tc_scores(z, W1, b1.reshape(1, _HH), ln_g.reshape(1, _HH),
                               ln_b.reshape(1, _HH), W2, b2.reshape(1, _N),
                               competence.reshape(1, _N),
                               activation_ema.reshape(1, _N))
    mask_flat, sel = _sc_topk()(padded.reshape(_B * _NP))
    mask = mask_flat.reshape(_B, _NP)[:, :_N]
    return mask, sel, exact
